# baseline (speedup 1.0000x reference)
import functools

import jax
import jax.numpy as jnp
from jax.experimental import pallas as pl
from jax.experimental.pallas import tpu as pltpu

_BN_EPS = 1e-5
_CIN = 4
_COUT = 9
_TROWS = 8


def _wino_image(x_ref, wa_ref, wb_ref, y_s, g, b, even):
    h, w = x_ref.shape[2], x_ref.shape[3]
    n_strips = h // (2 * _TROWS)
    zrow = jnp.zeros((1, w), jnp.float32)
    zlane = jnp.zeros((_TROWS, 1), jnp.float32)

    def shl(a):
        return jnp.concatenate([a[:, 1:], zlane], axis=1)

    def shr(a):
        return jnp.concatenate([zlane, a[:, : w - 1]], axis=1)

    tot = [None] * _COUT
    tot2 = [None] * _COUT
    for s in range(n_strips):
        base = 2 * _TROWS * s
        ts = []
        for ci in range(_CIN):
            ev = x_ref[b, ci, base:base + 2 * _TROWS:2, :]
            od = x_ref[b, ci, base + 1:base + 2 * _TROWS + 1:2, :]
            if s == 0:
                om1 = jnp.concatenate(
                    [zrow, x_ref[b, ci, 1:2 * _TROWS - 1:2, :]], axis=0)
            else:
                om1 = x_ref[b, ci, base - 1:base + 2 * _TROWS - 1:2, :]
            if s == n_strips - 1:
                ep1 = jnp.concatenate(
                    [x_ref[b, ci, base + 2:h:2, :], zrow], axis=0)
            else:
                ep1 = x_ref[b, ci, base + 2:base + 2 * _TROWS + 2:2, :]
            ts.append((om1 - od, ev + od, od - ev, ev - ep1))

        ma = [[None] * 4 for _ in range(_COUT)]
        mb = [[None] * 4 for _ in range(_COUT)]
        for u in range(4):
            for ci in range(_CIN):
                t = ts[ci][u]
                tp = shl(t)
                tm = shr(t)
                da = jnp.where(even, t + tp, t - tm)
                db = tm - tp
                for c in range(_COUT):
                    pa = da * wa_ref[c, ci, u]
                    pb = db * wb_ref[c, ci, u]
                    ma[c][u] = pa if ma[c][u] is None else ma[c][u] + pa
                    mb[c][u] = pb if mb[c][u] is None else mb[c][u] + pb

        for c in range(_COUT):
            ra0 = ma[c][0] + ma[c][1] + ma[c][2]
            ra1 = ma[c][1] - ma[c][2] - ma[c][3]
            rb0 = mb[c][0] + mb[c][1] + mb[c][2]
            rb1 = mb[c][1] - mb[c][2] - mb[c][3]
            ys = []
            for p, (ra, rb) in enumerate(((ra0, rb0), (ra1, rb1))):
                s1 = ra + rb
                y = jnp.where(even, s1 + shl(ra), shr(ra) - s1)
                y_s[g, c, p, _TROWS * s:_TROWS * (s + 1), :] = (
                    y.astype(jnp.bfloat16))
                ys.append(y)
            sy = ys[0] + ys[1]
            sy2 = ys[0] * ys[0] + ys[1] * ys[1]
            tot[c] = sy if tot[c] is None else tot[c] + sy
            tot2[c] = sy2 if tot2[c] is None else tot2[c] + sy2
    return tot, tot2


def _fused_kernel(x_ref, wa_ref, wb_ref, g_ref, b_ref, o_ref,
                  y_s, sum_s, sq_s, *, ib, n, h, w):
    i0 = pl.program_id(0)
    i1 = pl.program_id(1)
    lane = jax.lax.broadcasted_iota(jnp.int32, (_TROWS, w), 1)
    even = (lane % 2) == 0

    @pl.when(i0 == 0)
    def _conv_phase():
        @pl.when(i1 == 0)
        def _init():
            sum_s[...] = jnp.zeros_like(sum_s)
            sq_s[...] = jnp.zeros_like(sq_s)
        for b in range(ib):
            tot, tot2 = _wino_image(x_ref, wa_ref, wb_ref, y_s,
                                    i1 * ib + b, b, even)
            for c in range(_COUT):
                sum_s[c, :] += jnp.sum(tot[c], axis=0)
                sq_s[c, :] += jnp.sum(tot2[c], axis=0)

    @pl.when(i0 == 1)
    def _finish_phase():
        m_dim = float(n * h * w)
        mean = jnp.sum(sum_s[...], axis=1, keepdims=True) / m_dim
        ex2 = jnp.sum(sq_s[...], axis=1, keepdims=True) / m_dim
        var = jnp.maximum(ex2 - mean * mean, 0.0)
        scale = g_ref[...] * jax.lax.rsqrt(var + _BN_EPS)
        shift = b_ref[...] - mean * scale
        scale_b = jnp.broadcast_to(scale, (_COUT, w))
        shift_b = jnp.broadcast_to(shift, (_COUT, w))
        for b in range(ib):
            g = i1 * ib + b
            z = [y_s[g, c].astype(jnp.float32) * scale_b[c:c + 1, :]
                 + shift_b[c:c + 1, :] for c in range(_COUT)]
            m = z[0]
            for c in range(1, _COUT):
                m = jnp.maximum(m, z[c])
            e = [jnp.exp(z[c] - m) for c in range(_COUT)]
            d = e[0]
            for c in range(1, _COUT):
                d = d + e[c]
            r = pl.reciprocal(d, approx=True)
            for c in range(_COUT):
                v = e[c] * r
                o_ref[b, 0, c, 0:h:2, :] = v[0]
                o_ref[b, 0, c, 1:h:2, :] = v[1]


@jax.jit
def _adcnn(x, conv_w, gamma, beta):
    n, c_in, h, w = x.shape
    x = x.astype(jnp.float32)

    g_mat = jnp.array([[1.0, 0.0, 0.0],
                       [0.5, 0.5, 0.5],
                       [0.5, -0.5, 0.5],
                       [0.0, 0.0, 1.0]], jnp.float32)
    u_t = jnp.einsum("ua,cdab,vb->cduv", g_mat,
                     conv_w.astype(jnp.float32), g_mat)
    lane_even = (jnp.arange(w) % 2) == 0
    wa = jnp.where(lane_even[None, None, None, :],
                   u_t[..., 1:2], u_t[..., 2:3])
    wb = jnp.where(lane_even[None, None, None, :],
                   u_t[..., 0:1], u_t[..., 3:4])
    gam = gamma.astype(jnp.float32).reshape(_COUT, 1)
    bet = beta.astype(jnp.float32).reshape(_COUT, 1)

    ib = 2 if n % 2 == 0 else 1
    steps = n // ib

    out5 = pl.pallas_call(
        functools.partial(_fused_kernel, ib=ib, n=n, h=h, w=w),
        out_shape=jax.ShapeDtypeStruct((n, 1, _COUT, h, w), jnp.float32),
        grid=(2, steps),
        in_specs=[
            pl.BlockSpec((ib, c_in, h, w),
                         lambda i0, i1: (jnp.where(i0 == 0, i1, 0),
                                         0, 0, 0)),
            pl.BlockSpec((_COUT, _CIN, 4, w), lambda i0, i1: (0, 0, 0, 0)),
            pl.BlockSpec((_COUT, _CIN, 4, w), lambda i0, i1: (0, 0, 0, 0)),
            pl.BlockSpec((_COUT, 1), lambda i0, i1: (0, 0)),
            pl.BlockSpec((_COUT, 1), lambda i0, i1: (0, 0)),
        ],
        out_specs=pl.BlockSpec((ib, 1, _COUT, h, w),
                               lambda i0, i1: (jnp.where(i0 == 1, i1, 0),
                                               0, 0, 0, 0)),
        scratch_shapes=[
            pltpu.VMEM((n, _COUT, 2, h // 2, w), jnp.bfloat16),
            pltpu.VMEM((_COUT, w), jnp.float32),
            pltpu.VMEM((_COUT, w), jnp.float32),
        ],
        compiler_params=pltpu.CompilerParams(
            dimension_semantics=("arbitrary", "arbitrary")),
    )(x, wa, wb, gam, bet)

    return out5.reshape(n, 1, _COUT, h * w)


def kernel(x, conv_w, gamma, beta):
    return _adcnn(x, conv_w, gamma, beta)


# speedup vs baseline: 1.0181x; 1.0181x over previous
import functools

import jax
import jax.numpy as jnp
from jax.experimental import pallas as pl
from jax.experimental.pallas import tpu as pltpu

_BN_EPS = 1e-5
_CIN = 4
_COUT = 9
_TROWS = 8


def _wino_image(x_ref, wa_ref, wb_ref, y_ref, sum_ref, sq_ref, b, even):
    h, w = x_ref.shape[2], x_ref.shape[3]
    n_strips = h // (2 * _TROWS)
    zrow = jnp.zeros((1, w), jnp.float32)
    zlane = jnp.zeros((_TROWS, 1), jnp.float32)

    def shl(a):
        return jnp.concatenate([a[:, 1:], zlane], axis=1)

    def shr(a):
        return jnp.concatenate([zlane, a[:, : w - 1]], axis=1)

    tot = [None] * _COUT
    tot2 = [None] * _COUT
    for s in range(n_strips):
        base = 2 * _TROWS * s
        ts = []
        for ci in range(_CIN):
            ev = x_ref[b, ci, base:base + 2 * _TROWS:2, :]
            od = x_ref[b, ci, base + 1:base + 2 * _TROWS + 1:2, :]
            if s == 0:
                om1 = jnp.concatenate(
                    [zrow, x_ref[b, ci, 1:2 * _TROWS - 1:2, :]], axis=0)
            else:
                om1 = x_ref[b, ci, base - 1:base + 2 * _TROWS - 1:2, :]
            if s == n_strips - 1:
                ep1 = jnp.concatenate(
                    [x_ref[b, ci, base + 2:h:2, :], zrow], axis=0)
            else:
                ep1 = x_ref[b, ci, base + 2:base + 2 * _TROWS + 2:2, :]
            ts.append((om1 - od, ev + od, od - ev, ev - ep1))

        ma = [[None] * 4 for _ in range(_COUT)]
        mb = [[None] * 4 for _ in range(_COUT)]
        for u in range(4):
            for ci in range(_CIN):
                t = ts[ci][u]
                tp = shl(t)
                tm = shr(t)
                da = jnp.where(even, t + tp, t - tm)
                db = tm - tp
                for c in range(_COUT):
                    pa = da * wa_ref[c, ci, u]
                    pb = db * wb_ref[c, ci, u]
                    ma[c][u] = pa if ma[c][u] is None else ma[c][u] + pa
                    mb[c][u] = pb if mb[c][u] is None else mb[c][u] + pb

        for c in range(_COUT):
            ra0 = ma[c][0] + ma[c][1] + ma[c][2]
            ra1 = ma[c][1] - ma[c][2] - ma[c][3]
            rb0 = mb[c][0] + mb[c][1] + mb[c][2]
            rb1 = mb[c][1] - mb[c][2] - mb[c][3]
            ys = []
            for p, (ra, rb) in enumerate(((ra0, rb0), (ra1, rb1))):
                s1 = ra + rb
                y = jnp.where(even, s1 + shl(ra), shr(ra) - s1)
                y_ref[b, c, p, _TROWS * s:_TROWS * (s + 1), :] = (
                    y.astype(jnp.bfloat16))
                ys.append(y)
            sy = ys[0] + ys[1]
            sy2 = ys[0] * ys[0] + ys[1] * ys[1]
            tot[c] = sy if tot[c] is None else tot[c] + sy
            tot2[c] = sy2 if tot2[c] is None else tot2[c] + sy2
    for c in range(_COUT):
        sum_ref[b, c, :] = jnp.sum(tot[c], axis=0)
        sq_ref[b, c, :] = jnp.sum(tot2[c], axis=0)


def _conv_kernel(x_ref, wa_ref, wb_ref, y_ref, sum_ref, sq_ref, *, ib):
    w = x_ref.shape[3]
    lane = jax.lax.broadcasted_iota(jnp.int32, (_TROWS, w), 1)
    even = (lane % 2) == 0
    for b in range(ib):
        _wino_image(x_ref, wa_ref, wb_ref, y_ref, sum_ref, sq_ref, b, even)


def _finish_kernel(y_ref, sums_ref, sqs_ref, g_ref, b_ref, o_ref,
                   *, ib, m_dim):
    h = o_ref.shape[3]
    w = o_ref.shape[4]
    tot = jnp.sum(sums_ref[...], axis=0)
    tot2 = jnp.sum(sqs_ref[...], axis=0)
    mean = jnp.sum(tot, axis=1, keepdims=True) / m_dim
    ex2 = jnp.sum(tot2, axis=1, keepdims=True) / m_dim
    var = jnp.maximum(ex2 - mean * mean, 0.0)
    scale = g_ref[...] * jax.lax.rsqrt(var + _BN_EPS)
    shift = b_ref[...] - mean * scale
    scale_b = jnp.broadcast_to(scale, (_COUT, w))
    shift_b = jnp.broadcast_to(shift, (_COUT, w))
    for b in range(ib):
        z = [y_ref[b, c].astype(jnp.float32) * scale_b[c:c + 1, :]
             + shift_b[c:c + 1, :] for c in range(_COUT)]
        m = z[0]
        for c in range(1, _COUT):
            m = jnp.maximum(m, z[c])
        e = [jnp.exp(z[c] - m) for c in range(_COUT)]
        d = e[0]
        for c in range(1, _COUT):
            d = d + e[c]
        r = pl.reciprocal(d, approx=True)
        for c in range(_COUT):
            v = e[c] * r
            o_ref[b, 0, c, 0:h:2, :] = v[0]
            o_ref[b, 0, c, 1:h:2, :] = v[1]


@jax.jit
def _adcnn(x, conv_w, gamma, beta):
    n, c_in, h, w = x.shape
    x = x.astype(jnp.float32)

    g_mat = jnp.array([[1.0, 0.0, 0.0],
                       [0.5, 0.5, 0.5],
                       [0.5, -0.5, 0.5],
                       [0.0, 0.0, 1.0]], jnp.float32)
    u_t = jnp.einsum("ua,cdab,vb->cduv", g_mat,
                     conv_w.astype(jnp.float32), g_mat)
    lane_even = (jnp.arange(w) % 2) == 0
    wa = jnp.where(lane_even[None, None, None, :],
                   u_t[..., 1:2], u_t[..., 2:3])
    wb = jnp.where(lane_even[None, None, None, :],
                   u_t[..., 0:1], u_t[..., 3:4])

    ib = 2 if n % 2 == 0 else 1
    grid = (n // ib,)

    ybuf, sums, sqs = pl.pallas_call(
        functools.partial(_conv_kernel, ib=ib),
        out_shape=(
            jax.ShapeDtypeStruct((n, _COUT, 2, h // 2, w), jnp.bfloat16),
            jax.ShapeDtypeStruct((n, _COUT, w), jnp.float32),
            jax.ShapeDtypeStruct((n, _COUT, w), jnp.float32),
        ),
        grid=grid,
        in_specs=[
            pl.BlockSpec((ib, c_in, h, w), lambda i: (i, 0, 0, 0)),
            pl.BlockSpec((_COUT, _CIN, 4, w), lambda i: (0, 0, 0, 0)),
            pl.BlockSpec((_COUT, _CIN, 4, w), lambda i: (0, 0, 0, 0)),
        ],
        out_specs=(
            pl.BlockSpec((ib, _COUT, 2, h // 2, w),
                         lambda i: (i, 0, 0, 0, 0)),
            pl.BlockSpec((ib, _COUT, w), lambda i: (i, 0, 0)),
            pl.BlockSpec((ib, _COUT, w), lambda i: (i, 0, 0)),
        ),
        compiler_params=pltpu.CompilerParams(
            dimension_semantics=("parallel",)),
    )(x, wa, wb)

    gam = gamma.astype(jnp.float32).reshape(_COUT, 1)
    bet = beta.astype(jnp.float32).reshape(_COUT, 1)

    ib2 = 8 if n % 8 == 0 else 1
    out5 = pl.pallas_call(
        functools.partial(_finish_kernel, ib=ib2, m_dim=float(n * h * w)),
        out_shape=jax.ShapeDtypeStruct((n, 1, _COUT, h, w), jnp.float32),
        grid=(n // ib2,),
        in_specs=[
            pl.BlockSpec((ib2, _COUT, 2, h // 2, w),
                         lambda i: (i, 0, 0, 0, 0)),
            pl.BlockSpec((n, _COUT, w), lambda i: (0, 0, 0)),
            pl.BlockSpec((n, _COUT, w), lambda i: (0, 0, 0)),
            pl.BlockSpec((_COUT, 1), lambda i: (0, 0)),
            pl.BlockSpec((_COUT, 1), lambda i: (0, 0)),
        ],
        out_specs=pl.BlockSpec((ib2, 1, _COUT, h, w),
                               lambda i: (i, 0, 0, 0, 0)),
        compiler_params=pltpu.CompilerParams(
            dimension_semantics=("parallel",)),
    )(ybuf, sums, sqs, gam, bet)

    return out5.reshape(n, 1, _COUT, h * w)


def kernel(x, conv_w, gamma, beta):
    return _adcnn(x, conv_w, gamma, beta)


# speedup vs baseline: 1.1391x; 1.1188x over previous
import functools

import jax
import jax.numpy as jnp
from jax.experimental import pallas as pl
from jax.experimental.pallas import tpu as pltpu

_BN_EPS = 1e-5
_CIN = 4
_COUT = 9
_TROWS = 16


def _wino_image(x_ref, wa_ref, wb_ref, y_ref, sum_ref, sq_ref, b, even):
    h, w = x_ref.shape[2], x_ref.shape[3]
    n_strips = h // (2 * _TROWS)
    zrow = jnp.zeros((1, w), jnp.float32)
    zlane = jnp.zeros((_TROWS, 1), jnp.float32)

    def shl(a):
        return jnp.concatenate([a[:, 1:], zlane], axis=1)

    def shr(a):
        return jnp.concatenate([zlane, a[:, : w - 1]], axis=1)

    tot = [None] * _COUT
    tot2 = [None] * _COUT
    for s in range(n_strips):
        base = 2 * _TROWS * s
        ts = []
        for ci in range(_CIN):
            ev = x_ref[b, ci, base:base + 2 * _TROWS:2, :]
            od = x_ref[b, ci, base + 1:base + 2 * _TROWS + 1:2, :]
            if s == 0:
                om1 = jnp.concatenate(
                    [zrow, x_ref[b, ci, 1:2 * _TROWS - 1:2, :]], axis=0)
            else:
                om1 = x_ref[b, ci, base - 1:base + 2 * _TROWS - 1:2, :]
            if s == n_strips - 1:
                ep1 = jnp.concatenate(
                    [x_ref[b, ci, base + 2:h:2, :], zrow], axis=0)
            else:
                ep1 = x_ref[b, ci, base + 2:base + 2 * _TROWS + 2:2, :]
            ts.append((om1 - od, ev + od, od - ev, ev - ep1))

        ma = [[None] * 4 for _ in range(_COUT)]
        mb = [[None] * 4 for _ in range(_COUT)]
        for u in range(4):
            for ci in range(_CIN):
                t = ts[ci][u]
                tp = shl(t)
                tm = shr(t)
                da = jnp.where(even, t + tp, t - tm)
                db = tm - tp
                for c in range(_COUT):
                    pa = da * wa_ref[c, ci, u]
                    pb = db * wb_ref[c, ci, u]
                    ma[c][u] = pa if ma[c][u] is None else ma[c][u] + pa
                    mb[c][u] = pb if mb[c][u] is None else mb[c][u] + pb

        for c in range(_COUT):
            ra0 = ma[c][0] + ma[c][1] + ma[c][2]
            ra1 = ma[c][1] - ma[c][2] - ma[c][3]
            rb0 = mb[c][0] + mb[c][1] + mb[c][2]
            rb1 = mb[c][1] - mb[c][2] - mb[c][3]
            ys = []
            for p, (ra, rb) in enumerate(((ra0, rb0), (ra1, rb1))):
                s1 = ra + rb
                y = jnp.where(even, s1 + shl(ra), shr(ra) - s1)
                y_ref[b, c, p, _TROWS * s:_TROWS * (s + 1), :] = (
                    y.astype(jnp.bfloat16))
                ys.append(y)
            sy = ys[0] + ys[1]
            sy2 = ys[0] * ys[0] + ys[1] * ys[1]
            tot[c] = sy if tot[c] is None else tot[c] + sy
            tot2[c] = sy2 if tot2[c] is None else tot2[c] + sy2
    for c in range(_COUT):
        sum_ref[b, c, :] = jnp.sum(tot[c], axis=0)
        sq_ref[b, c, :] = jnp.sum(tot2[c], axis=0)


def _conv_kernel(x_ref, wa_ref, wb_ref, y_ref, sum_ref, sq_ref, *, ib):
    w = x_ref.shape[3]
    lane = jax.lax.broadcasted_iota(jnp.int32, (_TROWS, w), 1)
    even = (lane % 2) == 0
    for b in range(ib):
        _wino_image(x_ref, wa_ref, wb_ref, y_ref, sum_ref, sq_ref, b, even)


def _finish_kernel(y_ref, sums_ref, sqs_ref, g_ref, b_ref, o_ref,
                   aff_s, *, ib, m_dim):
    h = o_ref.shape[3]
    w = o_ref.shape[4]

    @pl.when(pl.program_id(0) == 0)
    def _fold_affine():
        tot = jnp.sum(sums_ref[...], axis=0)
        tot2 = jnp.sum(sqs_ref[...], axis=0)
        mean = jnp.sum(tot, axis=1, keepdims=True) / m_dim
        ex2 = jnp.sum(tot2, axis=1, keepdims=True) / m_dim
        var = jnp.maximum(ex2 - mean * mean, 0.0)
        scale = g_ref[...] * jax.lax.rsqrt(var + _BN_EPS)
        shift = b_ref[...] - mean * scale
        aff_s[0] = jnp.broadcast_to(scale, (_COUT, w))
        aff_s[1] = jnp.broadcast_to(shift, (_COUT, w))

    scale_b = aff_s[0]
    shift_b = aff_s[1]
    for b in range(ib):
        z = [y_ref[b, c].astype(jnp.float32) * scale_b[c:c + 1, :]
             + shift_b[c:c + 1, :] for c in range(_COUT)]
        m = z[0]
        for c in range(1, _COUT):
            m = jnp.maximum(m, z[c])
        e = [jnp.exp(z[c] - m) for c in range(_COUT)]
        d = e[0]
        for c in range(1, _COUT):
            d = d + e[c]
        r = pl.reciprocal(d, approx=True)
        for c in range(_COUT):
            v = e[c] * r
            o_ref[b, 0, c, 0:h:2, :] = v[0]
            o_ref[b, 0, c, 1:h:2, :] = v[1]


@jax.jit
def _adcnn(x, conv_w, gamma, beta):
    n, c_in, h, w = x.shape
    x = x.astype(jnp.float32)

    g_mat = jnp.array([[1.0, 0.0, 0.0],
                       [0.5, 0.5, 0.5],
                       [0.5, -0.5, 0.5],
                       [0.0, 0.0, 1.0]], jnp.float32)
    u_t = jnp.einsum("ua,cdab,vb->cduv", g_mat,
                     conv_w.astype(jnp.float32), g_mat)
    lane_even = (jnp.arange(w) % 2) == 0
    wa = jnp.where(lane_even[None, None, None, :],
                   u_t[..., 1:2], u_t[..., 2:3])
    wb = jnp.where(lane_even[None, None, None, :],
                   u_t[..., 0:1], u_t[..., 3:4])

    ib = 2 if n % 2 == 0 else 1
    grid = (n // ib,)

    ybuf, sums, sqs = pl.pallas_call(
        functools.partial(_conv_kernel, ib=ib),
        out_shape=(
            jax.ShapeDtypeStruct((n, _COUT, 2, h // 2, w), jnp.bfloat16),
            jax.ShapeDtypeStruct((n, _COUT, w), jnp.float32),
            jax.ShapeDtypeStruct((n, _COUT, w), jnp.float32),
        ),
        grid=grid,
        in_specs=[
            pl.BlockSpec((ib, c_in, h, w), lambda i: (i, 0, 0, 0)),
            pl.BlockSpec((_COUT, _CIN, 4, w), lambda i: (0, 0, 0, 0)),
            pl.BlockSpec((_COUT, _CIN, 4, w), lambda i: (0, 0, 0, 0)),
        ],
        out_specs=(
            pl.BlockSpec((ib, _COUT, 2, h // 2, w),
                         lambda i: (i, 0, 0, 0, 0)),
            pl.BlockSpec((ib, _COUT, w), lambda i: (i, 0, 0)),
            pl.BlockSpec((ib, _COUT, w), lambda i: (i, 0, 0)),
        ),
        compiler_params=pltpu.CompilerParams(
            dimension_semantics=("parallel",)),
    )(x, wa, wb)

    gam = gamma.astype(jnp.float32).reshape(_COUT, 1)
    bet = beta.astype(jnp.float32).reshape(_COUT, 1)

    ib2 = 8 if n % 8 == 0 else 1
    out5 = pl.pallas_call(
        functools.partial(_finish_kernel, ib=ib2, m_dim=float(n * h * w)),
        out_shape=jax.ShapeDtypeStruct((n, 1, _COUT, h, w), jnp.float32),
        grid=(n // ib2,),
        in_specs=[
            pl.BlockSpec((ib2, _COUT, 2, h // 2, w),
                         lambda i: (i, 0, 0, 0, 0)),
            pl.BlockSpec((n, _COUT, w), lambda i: (0, 0, 0)),
            pl.BlockSpec((n, _COUT, w), lambda i: (0, 0, 0)),
            pl.BlockSpec((_COUT, 1), lambda i: (0, 0)),
            pl.BlockSpec((_COUT, 1), lambda i: (0, 0)),
        ],
        out_specs=pl.BlockSpec((ib2, 1, _COUT, h, w),
                               lambda i: (i, 0, 0, 0, 0)),
        scratch_shapes=[pltpu.VMEM((2, _COUT, w), jnp.float32)],
        compiler_params=pltpu.CompilerParams(
            dimension_semantics=("arbitrary",)),
    )(ybuf, sums, sqs, gam, bet)

    return out5.reshape(n, 1, _COUT, h * w)


def kernel(x, conv_w, gamma, beta):
    return _adcnn(x, conv_w, gamma, beta)


# speedup vs baseline: 1.1501x; 1.0097x over previous
import functools

import jax
import jax.numpy as jnp
from jax.experimental import pallas as pl
from jax.experimental.pallas import tpu as pltpu

_BN_EPS = 1e-5
_CIN = 4
_COUT = 9
_TROWS = 16


def _wino_image(x_ref, wa_ref, wb_ref, y_ref, sum_ref, sq_ref, b, even):
    h, w = x_ref.shape[2], x_ref.shape[3]
    n_strips = h // (2 * _TROWS)
    zrow = jnp.zeros((1, w), jnp.float32)
    zlane = jnp.zeros((_TROWS, 1), jnp.float32)

    def shl(a):
        return jnp.concatenate([a[:, 1:], zlane], axis=1)

    def shr(a):
        return jnp.concatenate([zlane, a[:, : w - 1]], axis=1)

    tot = [None] * _COUT
    tot2 = [None] * _COUT
    for s in range(n_strips):
        base = 2 * _TROWS * s
        ts = []
        for ci in range(_CIN):
            ev = x_ref[b, ci, base:base + 2 * _TROWS:2, :]
            od = x_ref[b, ci, base + 1:base + 2 * _TROWS + 1:2, :]
            if s == 0:
                om1 = jnp.concatenate(
                    [zrow, x_ref[b, ci, 1:2 * _TROWS - 1:2, :]], axis=0)
            else:
                om1 = x_ref[b, ci, base - 1:base + 2 * _TROWS - 1:2, :]
            if s == n_strips - 1:
                ep1 = jnp.concatenate(
                    [x_ref[b, ci, base + 2:h:2, :], zrow], axis=0)
            else:
                ep1 = x_ref[b, ci, base + 2:base + 2 * _TROWS + 2:2, :]
            ts.append((om1 - od, ev + od, od - ev, ev - ep1))

        ma = [[None] * 4 for _ in range(_COUT)]
        mb = [[None] * 4 for _ in range(_COUT)]
        for u in range(4):
            for ci in range(_CIN):
                t = ts[ci][u]
                tp = shl(t)
                tm = shr(t)
                da = jnp.where(even, t + tp, t - tm)
                db = tm - tp
                for c in range(_COUT):
                    pa = da * wa_ref[c, ci, u]
                    pb = db * wb_ref[c, ci, u]
                    ma[c][u] = pa if ma[c][u] is None else ma[c][u] + pa
                    mb[c][u] = pb if mb[c][u] is None else mb[c][u] + pb

        for c in range(_COUT):
            ra0 = ma[c][0] + ma[c][1] + ma[c][2]
            ra1 = ma[c][1] - ma[c][2] - ma[c][3]
            rb0 = mb[c][0] + mb[c][1] + mb[c][2]
            rb1 = mb[c][1] - mb[c][2] - mb[c][3]
            ys = []
            for p, (ra, rb) in enumerate(((ra0, rb0), (ra1, rb1))):
                s1 = ra + rb
                y = jnp.where(even, s1 + shl(ra), shr(ra) - s1)
                y_ref[b, c, p, _TROWS * s:_TROWS * (s + 1), :] = (
                    y.astype(jnp.bfloat16))
                ys.append(y)
            sy = ys[0] + ys[1]
            sy2 = ys[0] * ys[0] + ys[1] * ys[1]
            tot[c] = sy if tot[c] is None else tot[c] + sy
            tot2[c] = sy2 if tot2[c] is None else tot2[c] + sy2
    for c in range(_COUT):
        sum_ref[b, c, :] = jnp.sum(tot[c], axis=0)
        sq_ref[b, c, :] = jnp.sum(tot2[c], axis=0)


def _conv_kernel(x_ref, wa_ref, wb_ref, y_ref, sum_ref, sq_ref, *, ib):
    w = x_ref.shape[3]
    lane = jax.lax.broadcasted_iota(jnp.int32, (_TROWS, w), 1)
    even = (lane % 2) == 0
    for b in range(ib):
        _wino_image(x_ref, wa_ref, wb_ref, y_ref, sum_ref, sq_ref, b, even)


def _finish_kernel(y_ref, sums_ref, sqs_ref, g_ref, b_ref, o_ref,
                   aff_s, *, ib, m_dim):
    h = o_ref.shape[3]
    w = o_ref.shape[4]

    @pl.when(pl.program_id(0) == 0)
    def _fold_affine():
        tot = jnp.sum(sums_ref[...], axis=0)
        tot2 = jnp.sum(sqs_ref[...], axis=0)
        mean = jnp.sum(tot, axis=1, keepdims=True) / m_dim
        ex2 = jnp.sum(tot2, axis=1, keepdims=True) / m_dim
        var = jnp.maximum(ex2 - mean * mean, 0.0)
        scale = g_ref[...] * jax.lax.rsqrt(var + _BN_EPS)
        shift = b_ref[...] - mean * scale
        aff_s[0] = jnp.broadcast_to(scale, (_COUT, w))
        aff_s[1] = jnp.broadcast_to(shift, (_COUT, w))

    scale_b = aff_s[0]
    shift_b = aff_s[1]
    for b in range(ib):
        z = [y_ref[b, c].astype(jnp.float32) * scale_b[c:c + 1, :]
             + shift_b[c:c + 1, :] for c in range(_COUT)]
        m = z[0]
        for c in range(1, _COUT):
            m = jnp.maximum(m, z[c])
        e = [jnp.exp(z[c] - m) for c in range(_COUT)]
        d = e[0]
        for c in range(1, _COUT):
            d = d + e[c]
        r = pl.reciprocal(d, approx=True)
        for c in range(_COUT):
            v = e[c] * r
            o_ref[b, 0, c, 0:h:2, :] = v[0]
            o_ref[b, 0, c, 1:h:2, :] = v[1]


@jax.jit
def _adcnn(x, conv_w, gamma, beta):
    n, c_in, h, w = x.shape
    x = x.astype(jnp.float32)

    g_mat = jnp.array([[1.0, 0.0, 0.0],
                       [0.5, 0.5, 0.5],
                       [0.5, -0.5, 0.5],
                       [0.0, 0.0, 1.0]], jnp.float32)
    u_t = jnp.einsum("ua,cdab,vb->cduv", g_mat,
                     conv_w.astype(jnp.float32), g_mat)
    lane_even = (jnp.arange(w) % 2) == 0
    wa = jnp.where(lane_even[None, None, None, :],
                   u_t[..., 1:2], u_t[..., 2:3])
    wb = jnp.where(lane_even[None, None, None, :],
                   u_t[..., 0:1], u_t[..., 3:4])

    ib = 4 if n % 4 == 0 else 1
    grid = (n // ib,)

    ybuf, sums, sqs = pl.pallas_call(
        functools.partial(_conv_kernel, ib=ib),
        out_shape=(
            jax.ShapeDtypeStruct((n, _COUT, 2, h // 2, w), jnp.bfloat16),
            jax.ShapeDtypeStruct((n, _COUT, w), jnp.float32),
            jax.ShapeDtypeStruct((n, _COUT, w), jnp.float32),
        ),
        grid=grid,
        in_specs=[
            pl.BlockSpec((ib, c_in, h, w), lambda i: (i, 0, 0, 0)),
            pl.BlockSpec((_COUT, _CIN, 4, w), lambda i: (0, 0, 0, 0)),
            pl.BlockSpec((_COUT, _CIN, 4, w), lambda i: (0, 0, 0, 0)),
        ],
        out_specs=(
            pl.BlockSpec((ib, _COUT, 2, h // 2, w),
                         lambda i: (i, 0, 0, 0, 0)),
            pl.BlockSpec((ib, _COUT, w), lambda i: (i, 0, 0)),
            pl.BlockSpec((ib, _COUT, w), lambda i: (i, 0, 0)),
        ),
        compiler_params=pltpu.CompilerParams(
            dimension_semantics=("parallel",)),
    )(x, wa, wb)

    gam = gamma.astype(jnp.float32).reshape(_COUT, 1)
    bet = beta.astype(jnp.float32).reshape(_COUT, 1)

    ib2 = 8 if n % 8 == 0 else 1
    out5 = pl.pallas_call(
        functools.partial(_finish_kernel, ib=ib2, m_dim=float(n * h * w)),
        out_shape=jax.ShapeDtypeStruct((n, 1, _COUT, h, w), jnp.float32),
        grid=(n // ib2,),
        in_specs=[
            pl.BlockSpec((ib2, _COUT, 2, h // 2, w),
                         lambda i: (i, 0, 0, 0, 0)),
            pl.BlockSpec((n, _COUT, w), lambda i: (0, 0, 0)),
            pl.BlockSpec((n, _COUT, w), lambda i: (0, 0, 0)),
            pl.BlockSpec((_COUT, 1), lambda i: (0, 0)),
            pl.BlockSpec((_COUT, 1), lambda i: (0, 0)),
        ],
        out_specs=pl.BlockSpec((ib2, 1, _COUT, h, w),
                               lambda i: (i, 0, 0, 0, 0)),
        scratch_shapes=[pltpu.VMEM((2, _COUT, w), jnp.float32)],
        compiler_params=pltpu.CompilerParams(
            dimension_semantics=("arbitrary",)),
    )(ybuf, sums, sqs, gam, bet)

    return out5.reshape(n, 1, _COUT, h * w)


def kernel(x, conv_w, gamma, beta):
    return _adcnn(x, conv_w, gamma, beta)


# speedup vs baseline: 1.1615x; 1.0099x over previous
import functools

import jax
import jax.numpy as jnp
from jax.experimental import pallas as pl
from jax.experimental.pallas import tpu as pltpu

_BN_EPS = 1e-5
_CIN = 4
_COUT = 9
_TROWS = 16


def _wino_image(x_ref, wa_ref, wb_ref, y_ref, sum_ref, sq_ref, b, even):
    h, w = x_ref.shape[2], x_ref.shape[3]
    n_strips = h // (2 * _TROWS)
    zrow = jnp.zeros((1, w), jnp.float32)
    zlane = jnp.zeros((_TROWS, 1), jnp.float32)

    def shl(a):
        return jnp.concatenate([a[:, 1:], zlane], axis=1)

    def shr(a):
        return jnp.concatenate([zlane, a[:, : w - 1]], axis=1)

    tot = [None] * _COUT
    tot2 = [None] * _COUT
    for s in range(n_strips):
        base = 2 * _TROWS * s
        ts = []
        for ci in range(_CIN):
            ev = x_ref[b, ci, base:base + 2 * _TROWS:2, :]
            od = x_ref[b, ci, base + 1:base + 2 * _TROWS + 1:2, :]
            if s == 0:
                om1 = jnp.concatenate(
                    [zrow, x_ref[b, ci, 1:2 * _TROWS - 1:2, :]], axis=0)
            else:
                om1 = x_ref[b, ci, base - 1:base + 2 * _TROWS - 1:2, :]
            if s == n_strips - 1:
                ep1 = jnp.concatenate(
                    [x_ref[b, ci, base + 2:h:2, :], zrow], axis=0)
            else:
                ep1 = x_ref[b, ci, base + 2:base + 2 * _TROWS + 2:2, :]
            ts.append((om1 - od, ev + od, od - ev, ev - ep1))

        ma = [[None] * 4 for _ in range(_COUT)]
        mb = [[None] * 4 for _ in range(_COUT)]
        for u in range(4):
            for ci in range(_CIN):
                t = ts[ci][u]
                tp = shl(t)
                tm = shr(t)
                da = jnp.where(even, t + tp, t - tm)
                db = tm - tp
                for c in range(_COUT):
                    pa = da * wa_ref[c, ci, u]
                    pb = db * wb_ref[c, ci, u]
                    ma[c][u] = pa if ma[c][u] is None else ma[c][u] + pa
                    mb[c][u] = pb if mb[c][u] is None else mb[c][u] + pb

        for c in range(_COUT):
            ra0 = ma[c][0] + ma[c][1] + ma[c][2]
            ra1 = ma[c][1] - ma[c][2] - ma[c][3]
            rb0 = mb[c][0] + mb[c][1] + mb[c][2]
            rb1 = mb[c][1] - mb[c][2] - mb[c][3]
            ys = []
            for p, (ra, rb) in enumerate(((ra0, rb0), (ra1, rb1))):
                s1 = ra + rb
                y = jnp.where(even, s1 + shl(ra), shr(ra) - s1)
                y_ref[b, c, p, _TROWS * s:_TROWS * (s + 1), :] = (
                    y.astype(jnp.bfloat16))
                ys.append(y)
            sy = ys[0] + ys[1]
            sy2 = ys[0] * ys[0] + ys[1] * ys[1]
            tot[c] = sy if tot[c] is None else tot[c] + sy
            tot2[c] = sy2 if tot2[c] is None else tot2[c] + sy2
    for c in range(_COUT):
        sum_ref[b, c, :] = jnp.sum(tot[c], axis=0)
        sq_ref[b, c, :] = jnp.sum(tot2[c], axis=0)


def _conv_kernel(x_ref, wa_ref, wb_ref, y_ref, sum_ref, sq_ref, *, ib):
    w = x_ref.shape[3]
    lane = jax.lax.broadcasted_iota(jnp.int32, (_TROWS, w), 1)
    even = (lane % 2) == 0
    for b in range(ib):
        _wino_image(x_ref, wa_ref, wb_ref, y_ref, sum_ref, sq_ref, b, even)


def _finish_kernel(y_ref, sums_ref, sqs_ref, g_ref, b_ref, o_ref,
                   aff_s, *, ib, m_dim):
    h = o_ref.shape[3]
    w = o_ref.shape[4]

    @pl.when(pl.program_id(0) == 0)
    def _fold_affine():
        tot = jnp.sum(sums_ref[...], axis=0)
        tot2 = jnp.sum(sqs_ref[...], axis=0)
        mean = jnp.sum(tot, axis=1, keepdims=True) / m_dim
        ex2 = jnp.sum(tot2, axis=1, keepdims=True) / m_dim
        var = jnp.maximum(ex2 - mean * mean, 0.0)
        scale = g_ref[...] * jax.lax.rsqrt(var + _BN_EPS)
        shift = b_ref[...] - mean * scale
        aff_s[0] = jnp.broadcast_to(scale, (_COUT, w))
        aff_s[1] = jnp.broadcast_to(shift, (_COUT, w))

    scale_b = aff_s[0]
    shift_b = aff_s[1]
    for b in range(ib):
        z = [y_ref[b, c].astype(jnp.float32) * scale_b[c:c + 1, :]
             + shift_b[c:c + 1, :] for c in range(_COUT)]
        m = z[0]
        for c in range(1, _COUT):
            m = jnp.maximum(m, z[c])
        e = [jnp.exp(z[c] - m) for c in range(_COUT)]
        d = e[0]
        for c in range(1, _COUT):
            d = d + e[c]
        r = pl.reciprocal(d, approx=True)
        for c in range(_COUT):
            v = e[c] * r
            o_ref[b, 0, c, 0:h:2, :] = v[0]
            o_ref[b, 0, c, 1:h:2, :] = v[1]


@jax.jit
def _adcnn(x, conv_w, gamma, beta):
    n, c_in, h, w = x.shape
    x = x.astype(jnp.float32)

    g_mat = jnp.array([[1.0, 0.0, 0.0],
                       [0.5, 0.5, 0.5],
                       [0.5, -0.5, 0.5],
                       [0.0, 0.0, 1.0]], jnp.float32)
    u_t = jnp.einsum("ua,cdab,vb->cduv", g_mat,
                     conv_w.astype(jnp.float32), g_mat)
    lane_even = (jnp.arange(w) % 2) == 0
    wa = jnp.where(lane_even[None, None, None, :],
                   u_t[..., 1:2], u_t[..., 2:3])
    wb = jnp.where(lane_even[None, None, None, :],
                   u_t[..., 0:1], u_t[..., 3:4])

    ib = 4 if n % 4 == 0 else 1
    grid = (n // ib,)

    ybuf, sums, sqs = pl.pallas_call(
        functools.partial(_conv_kernel, ib=ib),
        out_shape=(
            jax.ShapeDtypeStruct((n, _COUT, 2, h // 2, w), jnp.bfloat16),
            jax.ShapeDtypeStruct((n, _COUT, w), jnp.float32),
            jax.ShapeDtypeStruct((n, _COUT, w), jnp.float32),
        ),
        grid=grid,
        in_specs=[
            pl.BlockSpec((ib, c_in, h, w), lambda i: (i, 0, 0, 0)),
            pl.BlockSpec((_COUT, _CIN, 4, w), lambda i: (0, 0, 0, 0)),
            pl.BlockSpec((_COUT, _CIN, 4, w), lambda i: (0, 0, 0, 0)),
        ],
        out_specs=(
            pl.BlockSpec((ib, _COUT, 2, h // 2, w),
                         lambda i: (i, 0, 0, 0, 0)),
            pl.BlockSpec((ib, _COUT, w), lambda i: (i, 0, 0)),
            pl.BlockSpec((ib, _COUT, w), lambda i: (i, 0, 0)),
        ),
        compiler_params=pltpu.CompilerParams(
            dimension_semantics=("parallel",)),
    )(x, wa, wb)

    gam = gamma.astype(jnp.float32).reshape(_COUT, 1)
    bet = beta.astype(jnp.float32).reshape(_COUT, 1)

    ib2 = 16 if n % 16 == 0 else 1
    out5 = pl.pallas_call(
        functools.partial(_finish_kernel, ib=ib2, m_dim=float(n * h * w)),
        out_shape=jax.ShapeDtypeStruct((n, 1, _COUT, h, w), jnp.float32),
        grid=(n // ib2,),
        in_specs=[
            pl.BlockSpec((ib2, _COUT, 2, h // 2, w),
                         lambda i: (i, 0, 0, 0, 0)),
            pl.BlockSpec((n, _COUT, w), lambda i: (0, 0, 0)),
            pl.BlockSpec((n, _COUT, w), lambda i: (0, 0, 0)),
            pl.BlockSpec((_COUT, 1), lambda i: (0, 0)),
            pl.BlockSpec((_COUT, 1), lambda i: (0, 0)),
        ],
        out_specs=pl.BlockSpec((ib2, 1, _COUT, h, w),
                               lambda i: (i, 0, 0, 0, 0)),
        scratch_shapes=[pltpu.VMEM((2, _COUT, w), jnp.float32)],
        compiler_params=pltpu.CompilerParams(
            dimension_semantics=("arbitrary",)),
    )(ybuf, sums, sqs, gam, bet)

    return out5.reshape(n, 1, _COUT, h * w)


def kernel(x, conv_w, gamma, beta):
    return _adcnn(x, conv_w, gamma, beta)
